# Initial kernel scaffold; baseline (speedup 1.0000x reference)
import jax
import jax.numpy as jnp
from jax.experimental import pallas as pl


def kernel(nfeat, efeat, timestamps, W_self1, W_neigh1, b1, W_self2, W_neigh2, b2, freq, W_pred, b_pred, edge_index, batch_eids, neg_dst):
    raise NotImplementedError("write your pallas kernel here")


# speedup vs baseline: 2.0427x; 2.0427x over previous
import functools
import jax
import jax.numpy as jnp
from jax import lax
from jax.experimental import pallas as pl
from jax.experimental.pallas import tpu as pltpu
from jax.experimental.pallas import tpu_sc as plsc

N = 10000
E = 320000
DF = 128
DE = 16
H = 128
TD = 16
B = 1024
NNEG = 5

NC = 2
NS = 16
NW = NC * NS
C = 128
NCH = 80
GRP = 8
NGRP = NCH // GRP
EPW = NCH * C
EPAD = NW * EPW
NP = 10240
RPT = NP // NS
ZR = 128

_mesh = plsc.VectorSubcoreMesh(
    core_axis_name="c", subcore_axis_name="s", num_cores=NC, num_subcores=NS)


def _fill_vmem(buf, rows, width, val):
  def body(r, carry):
    for q in range(width // 16):
      buf[r, pl.ds(q * 16, 16)] = jnp.full((16,), val, jnp.float32)
    return carry
  lax.fori_loop(0, rows, body, 0)


def _worker_id():
  return lax.axis_index("c") * NS + lax.axis_index("s")


def _zero_stripes(subcore, zsrc, acc):
  for t in range(RPT // ZR):
    pltpu.sync_copy(zsrc, acc.at[pl.ds(subcore * RPT + t * ZR, ZR)])


def _chunk_loop(wid, idx_hbms, idx_vs, idx_flats, body):
  def group(t, carry):
    for hbm, v in zip(idx_hbms, idx_vs):
      pltpu.sync_copy(hbm.at[pl.ds(wid * NCH + t * GRP, GRP)], v)

    def sub(j, c2):
      for v, f in zip(idx_vs, idx_flats):
        for q in range(C // 16):
          f[pl.ds(q * 16, 16)] = v[j, pl.ds(q * 16, 16)]
      body(t * GRP + j, idx_flats)
      return c2
    lax.fori_loop(0, GRP, sub, 0)
    return carry
  lax.fori_loop(0, NGRP, group, 0)


@functools.partial(
    pl.kernel,
    out_type=jax.ShapeDtypeStruct((NC, NP, DF), jnp.float32),
    mesh=_mesh,
    scratch_types=[
        pltpu.VMEM((GRP, C), jnp.int32),
        pltpu.VMEM((GRP, C), jnp.int32),
        pltpu.VMEM((C,), jnp.int32),
        pltpu.VMEM((C,), jnp.int32),
        pltpu.VMEM((C, DF), jnp.float32),
        pltpu.VMEM_SHARED((NP, DF), jnp.float32),
    ],
)
def _sc_sums_nf(nf_hbm, src_hbm, dst_hbm, onf_hbm,
                idxs_v, idxd_v, fs_v, fd_v, rows_v, acc_nf):
  c = lax.axis_index("c")
  s = lax.axis_index("s")
  wid = c * NS + s
  _fill_vmem(rows_v, ZR, DF, 0.0)
  _zero_stripes(s, rows_v, acc_nf)
  plsc.subcore_barrier()

  def body(cidx, rows):
    sidx, didx = rows
    pltpu.sync_copy(nf_hbm.at[sidx], rows_v)
    pltpu.sync_copy(rows_v, acc_nf.at[didx], add=True)
  _chunk_loop(wid, [src_hbm, dst_hbm], [idxs_v, idxd_v], [fs_v, fd_v], body)
  plsc.subcore_barrier()

  stripe = pl.ds(s * RPT, RPT)
  pltpu.sync_copy(acc_nf.at[stripe], onf_hbm.at[c].at[stripe])


@functools.partial(
    pl.kernel,
    out_type=jax.ShapeDtypeStruct((NC, NP, H), jnp.float32),
    mesh=_mesh,
    scratch_types=[
        pltpu.VMEM((GRP, C), jnp.int32),
        pltpu.VMEM((C,), jnp.int32),
        pltpu.VMEM((C, DE), jnp.float32),
        pltpu.VMEM((C, H), jnp.float32),
        pltpu.VMEM_SHARED((NP, H), jnp.float32),
    ],
)
def _sc_sums_ef(ef_hbm, dst_hbm, oefc_hbm,
                idxd_v, fd_v, ef_v, combo_v, acc):
  c = lax.axis_index("c")
  s = lax.axis_index("s")
  wid = c * NS + s
  _fill_vmem(combo_v, ZR, H, 0.0)
  _zero_stripes(s, combo_v, acc)
  def setones(r, carry):
    combo_v[r, pl.ds(16, 16)] = jnp.full((16,), 1.0, jnp.float32)
    return carry
  lax.fori_loop(0, C, setones, 0)
  plsc.subcore_barrier()

  def body(cidx, rows):
    didx = rows[0]
    pltpu.sync_copy(ef_hbm.at[pl.ds(wid * EPW + cidx * C, C)], ef_v)
    def cprow(r, carry):
      combo_v[r, pl.ds(0, 16)] = ef_v[r, pl.ds(0, 16)]
      return carry
    lax.fori_loop(0, C, cprow, 0)
    pltpu.sync_copy(combo_v, acc.at[didx], add=True)
  _chunk_loop(wid, [dst_hbm], [idxd_v], [fd_v], body)
  plsc.subcore_barrier()

  stripe = pl.ds(s * RPT, RPT)
  pltpu.sync_copy(acc.at[stripe], oefc_hbm.at[c].at[stripe])


@functools.partial(
    pl.kernel,
    out_type=jax.ShapeDtypeStruct((NC, NP, H), jnp.float32),
    mesh=_mesh,
    scratch_types=[
        pltpu.VMEM((GRP, C), jnp.int32),
        pltpu.VMEM((C,), jnp.int32),
        pltpu.VMEM((C, H), jnp.float32),
        pltpu.VMEM_SHARED((NP, H), jnp.float32),
    ],
)
def _sc_cnt(idx_hbm, ocnt_hbm, idx_v, fidx_v, ones_v, acc):
  c = lax.axis_index("c")
  s = lax.axis_index("s")
  wid = c * NS + s
  _fill_vmem(ones_v, ZR, H, 0.0)
  _zero_stripes(s, ones_v, acc)
  _fill_vmem(ones_v, C, H, 1.0)
  plsc.subcore_barrier()

  def body(cidx, rows):
    pltpu.sync_copy(ones_v, acc.at[rows[0]], add=True)
  _chunk_loop(wid, [idx_hbm], [idx_v], [fidx_v], body)
  plsc.subcore_barrier()

  stripe = pl.ds(s * RPT, RPT)
  pltpu.sync_copy(acc.at[stripe], ocnt_hbm.at[c].at[stripe])


@functools.partial(
    pl.kernel,
    out_type=(
        jax.ShapeDtypeStruct((EPAD, H), jnp.float32),
        jax.ShapeDtypeStruct((EPAD, H), jnp.float32),
        jax.ShapeDtypeStruct((NC, NP, H), jnp.float32),
    ),
    mesh=_mesh,
    scratch_types=[
        pltpu.VMEM((GRP, C), jnp.int32),
        pltpu.VMEM((GRP, C), jnp.int32),
        pltpu.VMEM((C,), jnp.int32),
        pltpu.VMEM((C,), jnp.int32),
        pltpu.VMEM((C, H), jnp.float32),
        pltpu.VMEM((C, H), jnp.float32),
        pltpu.VMEM_SHARED((NP, H), jnp.float32),
    ],
)
def _sc_layer1(g1_hbm, ef1_hbm, src_hbm, dst_hbm,
               s1_hbm, d1_hbm, oagg_hbm,
               idxs_v, idxd_v, fs_v, fd_v, work_v, ef_v, acc):
  c = lax.axis_index("c")
  s = lax.axis_index("s")
  wid = c * NS + s
  _fill_vmem(work_v, ZR, H, 0.0)
  _zero_stripes(s, work_v, acc)
  plsc.subcore_barrier()

  def relu_add(r, rc):
    for q in range(H // 16):
      sl = pl.ds(q * 16, 16)
      work_v[r, sl] = jnp.maximum(work_v[r, sl] + ef_v[r, sl], 0.0)
    return rc

  def body(cidx, rows):
    sidx, didx = rows
    erow = pl.ds(wid * EPW + cidx * C, C)
    pltpu.sync_copy(ef1_hbm.at[erow], ef_v)
    pltpu.sync_copy(g1_hbm.at[sidx], work_v)
    lax.fori_loop(0, C, relu_add, 0)
    pltpu.sync_copy(work_v, s1_hbm.at[erow])
    pltpu.sync_copy(work_v, acc.at[didx], add=True)
    pltpu.sync_copy(g1_hbm.at[didx], work_v)
    lax.fori_loop(0, C, relu_add, 0)
    pltpu.sync_copy(work_v, d1_hbm.at[erow])
  _chunk_loop(wid, [src_hbm, dst_hbm], [idxs_v, idxd_v], [fs_v, fd_v], body)
  plsc.subcore_barrier()

  stripe = pl.ds(s * RPT, RPT)
  pltpu.sync_copy(acc.at[stripe], oagg_hbm.at[c].at[stripe])


@functools.partial(
    pl.kernel,
    out_type=jax.ShapeDtypeStruct((NC, NP, H), jnp.float32),
    mesh=_mesh,
    scratch_types=[
        pltpu.VMEM((GRP, C), jnp.int32),
        pltpu.VMEM((C,), jnp.int32),
        pltpu.VMEM((C, H), jnp.float32),
        pltpu.VMEM((C, H), jnp.float32),
        pltpu.VMEM_SHARED((NP, H), jnp.float32),
    ],
)
def _sc_seg(y_hbm, g2_hbm, idx_hbm, osum_hbm, idx_v, fidx_v, work_v, y_v, acc):
  c = lax.axis_index("c")
  s = lax.axis_index("s")
  wid = c * NS + s
  _fill_vmem(work_v, ZR, H, 0.0)
  _zero_stripes(s, work_v, acc)
  plsc.subcore_barrier()

  def relu_add(r, rc):
    for q in range(H // 16):
      sl = pl.ds(q * 16, 16)
      work_v[r, sl] = jnp.maximum(work_v[r, sl] + y_v[r, sl], 0.0)
    return rc

  def body(cidx, rows):
    idx = rows[0]
    pltpu.sync_copy(y_hbm.at[pl.ds(wid * EPW + cidx * C, C)], y_v)
    pltpu.sync_copy(g2_hbm.at[idx], work_v)
    lax.fori_loop(0, C, relu_add, 0)
    pltpu.sync_copy(work_v, acc.at[idx], add=True)
  _chunk_loop(wid, [idx_hbm], [idx_v], [fidx_v], body)
  plsc.subcore_barrier()

  stripe = pl.ds(s * RPT, RPT)
  pltpu.sync_copy(acc.at[stripe], osum_hbm.at[c].at[stripe])


BPW = B // NW
GPW = (B * NNEG) // NW

@functools.partial(
    pl.kernel,
    out_type=(
        jax.ShapeDtypeStruct((B,), jnp.float32),
        jax.ShapeDtypeStruct((B,), jnp.float32),
        jax.ShapeDtypeStruct((B,), jnp.float32),
        jax.ShapeDtypeStruct((B * NNEG,), jnp.float32),
    ),
    mesh=_mesh,
    scratch_types=[
        pltpu.VMEM((BPW,), jnp.int32),
        pltpu.VMEM((BPW,), jnp.int32),
        pltpu.VMEM((BPW,), jnp.int32),
        pltpu.VMEM((BPW,), jnp.float32),
        pltpu.VMEM((BPW,), jnp.float32),
        pltpu.VMEM((BPW,), jnp.float32),
        pltpu.VMEM((GPW,), jnp.int32),
        pltpu.VMEM((GPW,), jnp.float32),
    ],
)
def _sc_batch(src_hbm, dst_hbm, ts_hbm, be_hbm, neg_hbm,
              us_hbm, ud_hbm,
              ous_hbm, oud_hbm, ot_hbm, oneg_hbm,
              be_v, bs_v, bd_v,
              bus_v, bud_v, bt_v, nidx_v, nud_v):
  c = lax.axis_index("c")
  s = lax.axis_index("s")
  wid = c * NS + s
  pltpu.sync_copy(be_hbm.at[pl.ds(wid * BPW, BPW)], be_v)
  pltpu.sync_copy(neg_hbm.at[pl.ds(wid * GPW, GPW)], nidx_v)

  pltpu.sync_copy(src_hbm.at[be_v], bs_v)
  pltpu.sync_copy(dst_hbm.at[be_v], bd_v)
  pltpu.sync_copy(ts_hbm.at[be_v], bt_v)
  pltpu.sync_copy(us_hbm.at[bs_v], bus_v)
  pltpu.sync_copy(ud_hbm.at[bd_v], bud_v)
  pltpu.sync_copy(ud_hbm.at[nidx_v.at[pl.ds(0, GPW // 2)]],
                  nud_v.at[pl.ds(0, GPW // 2)])
  pltpu.sync_copy(ud_hbm.at[nidx_v.at[pl.ds(GPW // 2, GPW // 2)]],
                  nud_v.at[pl.ds(GPW // 2, GPW // 2)])

  pltpu.sync_copy(bus_v, ous_hbm.at[pl.ds(wid * BPW, BPW)])
  pltpu.sync_copy(bud_v, oud_hbm.at[pl.ds(wid * BPW, BPW)])
  pltpu.sync_copy(bt_v, ot_hbm.at[pl.ds(wid * BPW, BPW)])
  pltpu.sync_copy(nud_v, oneg_hbm.at[pl.ds(wid * GPW, GPW)])


NBLK = 1000
EBLK = 4096
MBLK = 2048


def _tc_g1_body(nf, snf, sefc, wsa, wna, wnb, b1, og1, ocd):
  cd = jnp.maximum(sefc[0, :, 16] + sefc[1, :, 16], 1.0)
  inv = (1.0 / cd)[:, None]
  agg_nf = (snf[0] + snf[1]) * inv
  agg_ef = (sefc[0, :, 0:16] + sefc[1, :, 0:16]) * inv
  g1 = jnp.dot(nf[...], wsa[...], preferred_element_type=jnp.float32)
  g1 += jnp.dot(agg_nf, wna[...], preferred_element_type=jnp.float32)
  g1 += jnp.dot(agg_ef, wnb[...], preferred_element_type=jnp.float32)
  og1[...] = g1 + b1[...]
  ocd[0, 0, :] = cd


def _tc_ef1_body(ef, w, out):
  out[...] = jnp.dot(ef[...], w[...], preferred_element_type=jnp.float32)


def _tc_g2_body(sagg, cd, wn2, b2, out):
  agg = (sagg[0] + sagg[1]) * (1.0 / jnp.maximum(cd[0, 0, :], 1.0))[:, None]
  out[...] = jnp.dot(agg, wn2[...], preferred_element_type=jnp.float32) + b2[...]


def _tc_mm2_body(s1, d1, w, oys, oyd):
  oys[...] = jnp.dot(s1[...], w[...], preferred_element_type=jnp.float32)
  oyd[...] = jnp.dot(d1[...], w[...], preferred_element_type=jnp.float32)


def _tc_u_body(uss, uds, scs, cd, w1, w2, ous, oud):
  cs = jnp.maximum(scs[0, :, 0] + scs[1, :, 0], 1.0)
  us = jnp.dot(uss[0] + uss[1], w1[...], preferred_element_type=jnp.float32)
  ud = jnp.dot(uds[0] + uds[1], w2[...], preferred_element_type=jnp.float32)
  ous[0, 0, :] = us[:, 0] / cs
  oud[0, 0, :] = ud[:, 0] / jnp.maximum(cd[0, 0, :], 1.0)


def _softplus(x):
  return jnp.maximum(x, 0.0) + jnp.log1p(jnp.exp(-jnp.abs(x)))


def _tc_loss_body(usb, udb, tb, udneg, freq, w3, bp, out):
  te = jnp.cos(tb[...][:, None] * freq[...])
  tw = jnp.dot(te, w3[...], preferred_element_type=jnp.float32)[:, 0]
  base = usb[...] + tw + bp[0, 0]
  pos = base + udb[...]
  neg = base[None, :] + udneg[...]
  loss = jnp.mean(_softplus(-pos)) + jnp.mean(_softplus(neg))
  out[...] = jnp.broadcast_to(loss, (1, 1))


def kernel(nfeat, efeat, timestamps, W_self1, W_neigh1, b1, W_self2,
           W_neigh2, b2, freq, W_pred, b_pred, edge_index, batch_eids,
           neg_dst):
  f32 = jnp.float32
  src = edge_index[0]
  dst = edge_index[1]
  pad_i = jnp.full((EPAD - E,), N, jnp.int32)
  src_p = jnp.concatenate([src, pad_i]).reshape(NW * NCH, C)
  dst_p = jnp.concatenate([dst, pad_i]).reshape(NW * NCH, C)
  nf_p = jnp.concatenate([nfeat, jnp.zeros((NP - N, DF), f32)])
  ef_p = jnp.concatenate([efeat, jnp.zeros((EPAD - E, DE), f32)])

  snf = _sc_sums_nf(nf_p, src_p, dst_p)
  sefc = _sc_sums_ef(ef_p, dst_p)
  scs = _sc_cnt(src_p)

  g1, cdeg = pl.pallas_call(
      _tc_g1_body,
      grid=(N // NBLK,),
      in_specs=[
          pl.BlockSpec((NBLK, DF), lambda i: (i, 0)),
          pl.BlockSpec((NC, NBLK, DF), lambda i: (0, i, 0)),
          pl.BlockSpec((NC, NBLK, H), lambda i: (0, i, 0)),
          pl.BlockSpec((DF, H), lambda i: (0, 0)),
          pl.BlockSpec((DF, H), lambda i: (0, 0)),
          pl.BlockSpec((DE, H), lambda i: (0, 0)),
          pl.BlockSpec((1, H), lambda i: (0, 0)),
      ],
      out_specs=[
          pl.BlockSpec((NBLK, H), lambda i: (i, 0)),
          pl.BlockSpec((1, 1, NBLK), lambda i: (i, 0, 0)),
      ],
      out_shape=[
          jax.ShapeDtypeStruct((N, H), f32),
          jax.ShapeDtypeStruct((N // NBLK, 1, NBLK), f32),
      ],
  )(nfeat, snf, sefc, W_self1[:DF], W_neigh1[:DF], W_neigh1[DF:],
    b1.reshape(1, H))
  g1_p = jnp.concatenate([g1, jnp.zeros((NP - N, H), f32)])

  ef1 = pl.pallas_call(
      _tc_ef1_body,
      grid=(EPAD // EBLK,),
      in_specs=[
          pl.BlockSpec((EBLK, DE), lambda i: (i, 0)),
          pl.BlockSpec((DE, H), lambda i: (0, 0)),
      ],
      out_specs=pl.BlockSpec((EBLK, H), lambda i: (i, 0)),
      out_shape=jax.ShapeDtypeStruct((EPAD, H), f32),
  )(ef_p, W_self1[DF:])

  s1, d1, sagg2 = _sc_layer1(g1_p, ef1, src_p, dst_p)

  g2 = pl.pallas_call(
      _tc_g2_body,
      grid=(N // NBLK,),
      in_specs=[
          pl.BlockSpec((NC, NBLK, H), lambda i: (0, i, 0)),
          pl.BlockSpec((1, 1, NBLK), lambda i: (i, 0, 0)),
          pl.BlockSpec((H, H), lambda i: (0, 0)),
          pl.BlockSpec((1, H), lambda i: (0, 0)),
      ],
      out_specs=pl.BlockSpec((NBLK, H), lambda i: (i, 0)),
      out_shape=jax.ShapeDtypeStruct((N, H), f32),
  )(sagg2, cdeg, W_neigh2, b2.reshape(1, H))
  g2_p = jnp.concatenate([g2, jnp.zeros((NP - N, H), f32)])

  ys, yd = pl.pallas_call(
      _tc_mm2_body,
      grid=(EPAD // MBLK,),
      in_specs=[
          pl.BlockSpec((MBLK, H), lambda i: (i, 0)),
          pl.BlockSpec((MBLK, H), lambda i: (i, 0)),
          pl.BlockSpec((H, H), lambda i: (0, 0)),
      ],
      out_specs=[
          pl.BlockSpec((MBLK, H), lambda i: (i, 0)),
          pl.BlockSpec((MBLK, H), lambda i: (i, 0)),
      ],
      out_shape=[
          jax.ShapeDtypeStruct((EPAD, H), f32),
          jax.ShapeDtypeStruct((EPAD, H), f32),
      ],
  )(s1, d1, W_self2)

  uss = _sc_seg(ys, g2_p, src_p)
  uds = _sc_seg(yd, g2_p, dst_p)

  u_s, u_d = pl.pallas_call(
      _tc_u_body,
      grid=(N // NBLK,),
      in_specs=[
          pl.BlockSpec((NC, NBLK, H), lambda i: (0, i, 0)),
          pl.BlockSpec((NC, NBLK, H), lambda i: (0, i, 0)),
          pl.BlockSpec((NC, NBLK, H), lambda i: (0, i, 0)),
          pl.BlockSpec((1, 1, NBLK), lambda i: (i, 0, 0)),
          pl.BlockSpec((H, 1), lambda i: (0, 0)),
          pl.BlockSpec((H, 1), lambda i: (0, 0)),
      ],
      out_specs=[
          pl.BlockSpec((1, 1, NBLK), lambda i: (i, 0, 0)),
          pl.BlockSpec((1, 1, NBLK), lambda i: (i, 0, 0)),
      ],
      out_shape=[
          jax.ShapeDtypeStruct((N // NBLK, 1, NBLK), f32),
          jax.ShapeDtypeStruct((N // NBLK, 1, NBLK), f32),
      ],
  )(uss, uds, scs, cdeg, W_pred[:H], W_pred[H:2 * H])

  usb, udb, tb, udneg = _sc_batch(src, dst, timestamps, batch_eids, neg_dst,
                                  u_s.reshape(N), u_d.reshape(N))

  loss = pl.pallas_call(
      _tc_loss_body,
      in_specs=[
          pl.BlockSpec((B,), lambda: (0,)),
          pl.BlockSpec((B,), lambda: (0,)),
          pl.BlockSpec((B,), lambda: (0,)),
          pl.BlockSpec((NNEG, B), lambda: (0, 0)),
          pl.BlockSpec((1, TD), lambda: (0, 0)),
          pl.BlockSpec((TD, 1), lambda: (0, 0)),
          pl.BlockSpec((1, 1), lambda: (0, 0)),
      ],
      out_specs=pl.BlockSpec((1, 1), lambda: (0, 0)),
      out_shape=jax.ShapeDtypeStruct((1, 1), f32),
  )(usb, udb, tb, udneg.reshape(NNEG, B), freq.reshape(1, TD),
    W_pred[2 * H:], b_pred.reshape(1, 1))

  return loss.reshape(())
